# Initial kernel scaffold; baseline (speedup 1.0000x reference)
import jax
import jax.numpy as jnp
from jax.experimental import pallas as pl


def kernel(inputs, we):
    raise NotImplementedError("write your pallas kernel here")


# speedup vs baseline: 3.1753x; 3.1753x over previous
import functools

import jax
import jax.numpy as jnp
from jax import lax
from jax.experimental import pallas as pl
from jax.experimental.pallas import tpu as pltpu
from jax.experimental.pallas import tpu_sc as plsc

NUM_CORES = 2
NUM_SUBCORES = 16
NW = NUM_CORES * NUM_SUBCORES
LANES = 16

G = 128


def _sc_body(idx_hbm, table_hbm, out_hbm, idx_v, rows_v, out_v, sem,
             *, rows_per_w, d):
    groups = (2 * rows_per_w) // G
    out_rows = G // 2
    wid = lax.axis_index("s") * NUM_CORES + lax.axis_index("c")
    row_base = wid * rows_per_w

    pltpu.sync_copy(idx_hbm.at[wid], idx_v)

    def compute_row(i, carry):
        for s in range(d // LANES):
            sl = pl.ds(s * LANES, LANES)
            out_v[i, sl] = rows_v[2 * i, sl] + rows_v[2 * i + 1, sl]
        return carry

    def group_body(k, carry):
        pltpu.async_copy(table_hbm.at[idx_v.at[k]], rows_v, sem).wait()
        lax.fori_loop(0, out_rows, compute_row, 0, unroll=2)
        pltpu.sync_copy(out_v, out_hbm.at[pl.ds(row_base + k * out_rows,
                                                out_rows)])
        return carry

    lax.fori_loop(0, groups, group_body, 0)


@functools.partial(jax.jit, static_argnums=(2, 3))
def _sc_embed(idx2d, we, n_rows, d):
    rows_per_w = n_rows // NW
    body = functools.partial(_sc_body, rows_per_w=rows_per_w, d=d)
    groups = (2 * rows_per_w) // G
    k = pl.kernel(
        body,
        out_type=jax.ShapeDtypeStruct((n_rows, d), jnp.float32),
        mesh=plsc.VectorSubcoreMesh(core_axis_name="c", subcore_axis_name="s"),
        scratch_types=[
            pltpu.VMEM((groups, G), jnp.int32),
            pltpu.VMEM((G, d), jnp.float32),
            pltpu.VMEM((G // 2, d), jnp.float32),
            pltpu.SemaphoreType.DMA,
        ],
    )
    return k(idx2d, we)


def kernel(inputs, we):
    b, l, two = inputs.shape
    n_rows = b * l
    d = we.shape[1]
    idx2d = inputs.astype(jnp.int32).reshape(NW, n_rows * two // (NW * G), G)
    out = _sc_embed(idx2d, we, n_rows, d)
    return out.reshape(b, l, d)


# speedup vs baseline: 4.2055x; 1.3244x over previous
import functools

import jax
import jax.numpy as jnp
from jax import lax
from jax.experimental import pallas as pl
from jax.experimental.pallas import tpu as pltpu
from jax.experimental.pallas import tpu_sc as plsc

NUM_CORES = 2
NUM_SUBCORES = 16
NW = NUM_CORES * NUM_SUBCORES
LANES = 16

G = 128


def _sc_body(idx_hbm, table_hbm, out_hbm, idx_v, rows_v, out_v, gsem0, gsem1,
             *, rows_per_w, d):
    groups = (2 * rows_per_w) // G
    out_rows = G // 2
    wid = lax.axis_index("s") * NUM_CORES + lax.axis_index("c")
    row_base = wid * rows_per_w
    gsems = (gsem0, gsem1)

    pltpu.sync_copy(idx_hbm.at[wid], idx_v)

    def start_g(k, b):
        pltpu.async_copy(table_hbm.at[idx_v.at[k]], rows_v.at[b], gsems[b])

    def wait_g(b):
        pltpu.make_async_copy(table_hbm.at[pl.ds(0, G)], rows_v.at[b],
                              gsems[b]).wait()

    def compute_store(k, b):
        rref = rows_v.at[b]

        def compute_row(i, carry):
            for s in range(d // LANES):
                sl = pl.ds(s * LANES, LANES)
                out_v[i, sl] = rref[2 * i, sl] + rref[2 * i + 1, sl]
            return carry

        lax.fori_loop(0, out_rows, compute_row, 0, unroll=2)
        pltpu.sync_copy(out_v, out_hbm.at[pl.ds(row_base + k * out_rows,
                                                out_rows)])

    start_g(0, 0)

    def outer(kk, carry):
        k = 2 * kk
        wait_g(0)
        start_g(k + 1, 1)
        compute_store(k, 0)
        wait_g(1)
        start_g(k + 2, 0)
        compute_store(k + 1, 1)
        return carry

    lax.fori_loop(0, groups // 2 - 1, outer, 0)
    wait_g(0)
    start_g(groups - 1, 1)
    compute_store(groups - 2, 0)
    wait_g(1)
    compute_store(groups - 1, 1)


@functools.partial(jax.jit, static_argnums=(2, 3))
def _sc_embed(idx3d, we, n_rows, d):
    rows_per_w = n_rows // NW
    body = functools.partial(_sc_body, rows_per_w=rows_per_w, d=d)
    groups = (2 * rows_per_w) // G
    assert groups % 2 == 0 and groups >= 4
    k = pl.kernel(
        body,
        out_type=jax.ShapeDtypeStruct((n_rows, d), jnp.float32),
        mesh=plsc.VectorSubcoreMesh(core_axis_name="c", subcore_axis_name="s"),
        scratch_types=[
            pltpu.VMEM((groups, G), jnp.int32),
            pltpu.VMEM((2, G, d), jnp.float32),
            pltpu.VMEM((G // 2, d), jnp.float32),
            pltpu.SemaphoreType.DMA,
            pltpu.SemaphoreType.DMA,
        ],
    )
    return k(idx3d, we)


def kernel(inputs, we):
    b, l, two = inputs.shape
    n_rows = b * l
    d = we.shape[1]
    idx3d = inputs.astype(jnp.int32).reshape(NW, n_rows * two // (NW * G), G)
    out = _sc_embed(idx3d, we, n_rows, d)
    return out.reshape(b, l, d)


# speedup vs baseline: 6.1853x; 1.4708x over previous
import functools

import jax
import jax.numpy as jnp
from jax import lax
from jax.experimental import pallas as pl
from jax.experimental.pallas import tpu as pltpu
from jax.experimental.pallas import tpu_sc as plsc

NUM_CORES = 2
NUM_SUBCORES = 16
NW = NUM_CORES * NUM_SUBCORES
LANES = 16

G = 128


def _sc_body(idx_hbm, table_hbm, out_hbm, idx_v, rows_v, out_v, gsem0, gsem1,
             osem0, osem1, *, rows_per_w, d):
    groups = (2 * rows_per_w) // G
    out_rows = G // 2
    wid = lax.axis_index("s") * NUM_CORES + lax.axis_index("c")
    row_base = wid * rows_per_w
    gsems = (gsem0, gsem1)
    osems = (osem0, osem1)

    pltpu.sync_copy(idx_hbm.at[wid], idx_v)

    def start_g(k, b):
        pltpu.async_copy(table_hbm.at[idx_v.at[k]], rows_v.at[b], gsems[b])

    def wait_g(b):
        pltpu.make_async_copy(table_hbm.at[pl.ds(0, G)], rows_v.at[b],
                              gsems[b]).wait()

    def out_slice(k):
        return out_hbm.at[pl.ds(row_base + k * out_rows, out_rows)]

    def wait_o(b):
        pltpu.make_async_copy(out_v.at[b], out_slice(0), osems[b]).wait()

    def compute_store(k, b, first):
        rref = rows_v.at[b]
        oref = out_v.at[b]
        if not first:
            wait_o(b)

        @plsc.parallel_loop(0, out_rows, unroll=4)
        def compute_row(i):
            for s in range(d // LANES):
                sl = pl.ds(s * LANES, LANES)
                oref[i, sl] = rref[2 * i, sl] + rref[2 * i + 1, sl]

        pltpu.async_copy(oref, out_slice(k), osems[b])

    start_g(0, 0)
    wait_g(0)
    start_g(1, 1)
    compute_store(0, 0, True)
    wait_g(1)
    start_g(2, 0)
    compute_store(1, 1, True)

    def outer(kk, carry):
        k = 2 * kk
        wait_g(0)
        start_g(k + 1, 1)
        compute_store(k, 0, False)
        wait_g(1)
        start_g(k + 2, 0)
        compute_store(k + 1, 1, False)
        return carry

    lax.fori_loop(1, groups // 2 - 1, outer, 0)
    wait_g(0)
    start_g(groups - 1, 1)
    compute_store(groups - 2, 0, False)
    wait_g(1)
    compute_store(groups - 1, 1, False)
    wait_o(0)
    wait_o(1)


@functools.partial(jax.jit, static_argnums=(2, 3))
def _sc_embed(idx3d, we, n_rows, d):
    rows_per_w = n_rows // NW
    body = functools.partial(_sc_body, rows_per_w=rows_per_w, d=d)
    groups = (2 * rows_per_w) // G
    assert groups % 2 == 0 and groups >= 4
    k = pl.kernel(
        body,
        out_type=jax.ShapeDtypeStruct((n_rows, d), jnp.float32),
        mesh=plsc.VectorSubcoreMesh(core_axis_name="c", subcore_axis_name="s"),
        scratch_types=[
            pltpu.VMEM((groups, G), jnp.int32),
            pltpu.VMEM((2, G, d), jnp.float32),
            pltpu.VMEM((2, G // 2, d), jnp.float32),
            pltpu.SemaphoreType.DMA,
            pltpu.SemaphoreType.DMA,
            pltpu.SemaphoreType.DMA,
            pltpu.SemaphoreType.DMA,
        ],
    )
    return k(idx3d, we)


def kernel(inputs, we):
    b, l, two = inputs.shape
    n_rows = b * l
    d = we.shape[1]
    idx3d = inputs.astype(jnp.int32).reshape(NW, n_rows * two // (NW * G), G)
    out = _sc_embed(idx3d, we, n_rows, d)
    return out.reshape(b, l, d)


# speedup vs baseline: 12.7222x; 2.0568x over previous
import functools

import jax
import jax.numpy as jnp
from jax import lax
from jax.experimental import pallas as pl
from jax.experimental.pallas import tpu as pltpu
from jax.experimental.pallas import tpu_sc as plsc

NUM_CORES = 2
NUM_SUBCORES = 16
NW = NUM_CORES * NUM_SUBCORES
LANES = 16

G = 128


def _sc_body(idx_hbm, table_hbm, out_hbm, idx_v, rows_v, out_v, sidx_v,
             base_v, gsem0, gsem1, osem0, osem1,
             *, blocks_per_w, blocks_per_l, l_len, d):
    nsl = d // LANES
    wid = lax.axis_index("s") * NUM_CORES + lax.axis_index("c")
    blk0 = wid * blocks_per_w
    gsems = (gsem0, gsem1)
    osems = (osem0, osem1)

    pltpu.sync_copy(idx_hbm.at[wid], idx_v)

    for s in range(G // LANES):
        base_v[pl.ds(s * LANES, LANES)] = (
            lax.iota(jnp.int32, LANES) + s * LANES) * l_len

    def start_g(j, b):
        pltpu.async_copy(table_hbm.at[idx_v.at[2 * j]],
                         rows_v.at[b, pl.ds(0, G)], gsems[b])
        pltpu.async_copy(table_hbm.at[idx_v.at[2 * j + 1]],
                         rows_v.at[b, pl.ds(G, G)], gsems[b])

    def wait_g(b):
        pltpu.make_async_copy(table_hbm.at[pl.ds(0, 2 * G)], rows_v.at[b],
                              gsems[b]).wait()

    def wait_o(b):
        pltpu.make_async_copy(out_v.at[b], out_hbm.at[sidx_v.at[b]],
                              osems[b]).wait()

    def compute_store(j, b, first):
        if not first:
            wait_o(b)
        blk = blk0 + j
        l = blk // blocks_per_l
        tc = blk - l * blocks_per_l
        cst = tc * (G * l_len) + l
        for s in range(G // LANES):
            sl = pl.ds(s * LANES, LANES)
            sidx_v[b, sl] = base_v[sl] + cst
        rref = rows_v.at[b]
        oref = out_v.at[b]

        @plsc.parallel_loop(0, G, unroll=4)
        def compute_row(c):
            for s in range(nsl):
                sl = pl.ds(s * LANES, LANES)
                oref[c, sl] = rref[c, sl] + rref[G + c, sl]

        pltpu.async_copy(oref, out_hbm.at[sidx_v.at[b]], osems[b])

    start_g(0, 0)
    wait_g(0)
    start_g(1, 1)
    compute_store(0, 0, True)
    wait_g(1)
    start_g(2, 0)
    compute_store(1, 1, True)

    def outer(kk, carry):
        j = 2 * kk
        wait_g(0)
        start_g(j + 1, 1)
        compute_store(j, 0, False)
        wait_g(1)
        start_g(j + 2, 0)
        compute_store(j + 1, 1, False)
        return carry

    lax.fori_loop(1, blocks_per_w // 2 - 1, outer, 0)
    wait_g(0)
    start_g(blocks_per_w - 1, 1)
    compute_store(blocks_per_w - 2, 0, False)
    wait_g(1)
    compute_store(blocks_per_w - 1, 1, False)
    wait_o(0)
    wait_o(1)


@functools.partial(jax.jit, static_argnums=(2, 3, 4))
def _sc_embed(idx3d, we, n_rows, l_len, d):
    blocks = (2 * n_rows) // (2 * G)
    blocks_per_w = blocks // NW
    blocks_per_l = n_rows // (l_len * G)
    assert blocks_per_w % 2 == 0 and blocks_per_w >= 6
    body = functools.partial(_sc_body, blocks_per_w=blocks_per_w,
                             blocks_per_l=blocks_per_l, l_len=l_len, d=d)
    k = pl.kernel(
        body,
        out_type=jax.ShapeDtypeStruct((n_rows, d), jnp.float32),
        mesh=plsc.VectorSubcoreMesh(core_axis_name="c", subcore_axis_name="s"),
        scratch_types=[
            pltpu.VMEM((2 * blocks_per_w, G), jnp.int32),
            pltpu.VMEM((2, 2 * G, d), jnp.float32),
            pltpu.VMEM((2, G, d), jnp.float32),
            pltpu.VMEM((2, G), jnp.int32),
            pltpu.VMEM((G,), jnp.int32),
            pltpu.SemaphoreType.DMA,
            pltpu.SemaphoreType.DMA,
            pltpu.SemaphoreType.DMA,
            pltpu.SemaphoreType.DMA,
        ],
    )
    return k(idx3d, we)


def kernel(inputs, we):
    b, l_len, two = inputs.shape
    n_rows = b * l_len
    d = we.shape[1]
    blocks_per_l = b // G
    x8 = inputs.astype(jnp.int32).reshape(blocks_per_l, G, l_len, two)
    wphys = jnp.transpose(x8, (2, 0, 3, 1))
    idx3d = wphys.reshape(NW, (n_rows * two) // (NW * G), G)
    out = _sc_embed(idx3d, we, n_rows, l_len, d)
    return out.reshape(b, l_len, d)


# speedup vs baseline: 13.6053x; 1.0694x over previous
import functools

import jax
import jax.numpy as jnp
from jax import lax
from jax.experimental import pallas as pl
from jax.experimental.pallas import tpu as pltpu
from jax.experimental.pallas import tpu_sc as plsc

NUM_CORES = 2
NUM_SUBCORES = 16
NW = NUM_CORES * NUM_SUBCORES
LANES = 16

G = 128


def _sc_body(idx_hbm, table_hbm, out_hbm, idx_v, rows_v, out_v, sidx_v,
             base_v, gsem0, gsem1, osem0, osem1,
             *, blocks_per_w, blocks_per_l, l_len, d):
    nsl = d // LANES
    wid = lax.axis_index("s") * NUM_CORES + lax.axis_index("c")
    blk0 = wid * blocks_per_w
    gsems = (gsem0, gsem1)
    osems = (osem0, osem1)

    pltpu.sync_copy(idx_hbm.at[wid], idx_v)

    for s in range(G // LANES):
        base_v[pl.ds(s * LANES, LANES)] = (
            lax.iota(jnp.int32, LANES) + s * LANES) * l_len

    def start_g(j, b):
        pltpu.async_copy(table_hbm.at[idx_v.at[2 * j]],
                         rows_v.at[b, pl.ds(0, G)], gsems[b])
        pltpu.async_copy(table_hbm.at[idx_v.at[2 * j + 1]],
                         rows_v.at[b, pl.ds(G, G)], gsems[b])

    def wait_g(b):
        pltpu.make_async_copy(table_hbm.at[pl.ds(0, 2 * G)], rows_v.at[b],
                              gsems[b]).wait()

    def wait_o(b):
        pltpu.make_async_copy(out_v.at[b], out_hbm.at[sidx_v.at[b]],
                              osems[b]).wait()

    def compute_store(j, b, first):
        if not first:
            wait_o(b)
        blk = blk0 + j
        l = blk // blocks_per_l
        tc = blk - l * blocks_per_l
        cst = tc * (G * l_len) + l
        for s in range(G // LANES):
            sl = pl.ds(s * LANES, LANES)
            sidx_v[b, sl] = base_v[sl] + cst
        rref = rows_v.at[b]
        oref = out_v.at[b]

        @plsc.parallel_loop(0, G, unroll=4)
        def compute_row(c):
            for s in range(nsl):
                sl = pl.ds(s * LANES, LANES)
                oref[c, sl] = rref[c, sl] + rref[G + c, sl]

        pltpu.async_copy(oref, out_hbm.at[sidx_v.at[b]], osems[b])

    start_g(0, 0)
    start_g(1, 1)
    wait_g(0)
    compute_store(0, 0, True)
    start_g(2, 0)
    wait_g(1)
    compute_store(1, 1, True)

    def outer(kk, carry):
        j = 2 * kk
        start_g(j + 1, 1)
        wait_g(0)
        compute_store(j, 0, False)
        start_g(j + 2, 0)
        wait_g(1)
        compute_store(j + 1, 1, False)
        return carry

    lax.fori_loop(1, blocks_per_w // 2 - 1, outer, 0)
    start_g(blocks_per_w - 1, 1)
    wait_g(0)
    compute_store(blocks_per_w - 2, 0, False)
    wait_g(1)
    compute_store(blocks_per_w - 1, 1, False)
    wait_o(0)
    wait_o(1)


@functools.partial(jax.jit, static_argnums=(2, 3, 4))
def _sc_embed(idx3d, we, n_rows, l_len, d):
    blocks = (2 * n_rows) // (2 * G)
    blocks_per_w = blocks // NW
    blocks_per_l = n_rows // (l_len * G)
    assert blocks_per_w % 2 == 0 and blocks_per_w >= 6
    body = functools.partial(_sc_body, blocks_per_w=blocks_per_w,
                             blocks_per_l=blocks_per_l, l_len=l_len, d=d)
    k = pl.kernel(
        body,
        out_type=jax.ShapeDtypeStruct((n_rows, d), jnp.float32),
        mesh=plsc.VectorSubcoreMesh(core_axis_name="c", subcore_axis_name="s"),
        scratch_types=[
            pltpu.VMEM((2 * blocks_per_w, G), jnp.int32),
            pltpu.VMEM((2, 2 * G, d), jnp.float32),
            pltpu.VMEM((2, G, d), jnp.float32),
            pltpu.VMEM((2, G), jnp.int32),
            pltpu.VMEM((G,), jnp.int32),
            pltpu.SemaphoreType.DMA,
            pltpu.SemaphoreType.DMA,
            pltpu.SemaphoreType.DMA,
            pltpu.SemaphoreType.DMA,
        ],
    )
    return k(idx3d, we)


def kernel(inputs, we):
    b, l_len, two = inputs.shape
    n_rows = b * l_len
    d = we.shape[1]
    blocks_per_l = b // G
    x8 = inputs.astype(jnp.int32).reshape(blocks_per_l, G, l_len, two)
    wphys = jnp.transpose(x8, (2, 0, 3, 1))
    idx3d = wphys.reshape(NW, (n_rows * two) // (NW * G), G)
    out = _sc_embed(idx3d, we, n_rows, l_len, d)
    return out.reshape(b, l_len, d)


# speedup vs baseline: 13.6223x; 1.0012x over previous
import functools

import jax
import jax.numpy as jnp
from jax import lax
from jax.experimental import pallas as pl
from jax.experimental.pallas import tpu as pltpu
from jax.experimental.pallas import tpu_sc as plsc

NUM_CORES = 2
NUM_SUBCORES = 16
NW = NUM_CORES * NUM_SUBCORES
LANES = 16

G = 128
H = G // 2
DEPTH = 3


def _sc_body(idx_hbm, table_hbm, out_hbm, idx_v, rows_v, out_v, sidx_v,
             base_v, gsems, osems, *, blocks_per_w, blocks_per_l, l_len, d):
    nsl = d // LANES
    wid = lax.axis_index("s") * NUM_CORES + lax.axis_index("c")
    blk0 = wid * blocks_per_w

    pltpu.sync_copy(idx_hbm.at[wid], idx_v)

    for s in range(H // LANES):
        base_v[pl.ds(s * LANES, LANES)] = (
            lax.iota(jnp.int32, LANES) + s * LANES) * l_len

    def start_u(u, b):
        pltpu.async_copy(table_hbm.at[idx_v.at[u]], rows_v.at[b], gsems[b])

    def wait_u(b):
        pltpu.make_async_copy(table_hbm.at[pl.ds(0, G)], rows_v.at[b],
                              gsems[b]).wait()

    def wait_o(h):
        pltpu.make_async_copy(out_v.at[h], out_hbm.at[sidx_v.at[h]],
                              osems[h]).wait()

    def do_block(j, p, first):
        b0 = (2 * p) % (2 * DEPTH)
        b1 = b0 + 1
        wait_u(b0)
        wait_u(b1)
        blk = blk0 + j
        l = blk // blocks_per_l
        tc = blk - l * blocks_per_l
        cst = tc * (G * l_len) + l
        rref0 = rows_v.at[b0]
        rref1 = rows_v.at[b1]
        for h in range(2):
            if not first:
                wait_o(h)
            csth = cst + (h * H) * l_len
            for s in range(H // LANES):
                sl = pl.ds(s * LANES, LANES)
                sidx_v[h, sl] = base_v[sl] + csth
            oref = out_v.at[h]

            @plsc.parallel_loop(0, H, unroll=4)
            def compute_row(c):
                for s in range(nsl):
                    sl = pl.ds(s * LANES, LANES)
                    oref[c, sl] = rref0[h * H + c, sl] + rref1[h * H + c, sl]

            pltpu.async_copy(oref, out_hbm.at[sidx_v.at[h]], osems[h])

    def start_blk(m, p):
        @pl.when(m < blocks_per_w)
        def _():
            start_u(2 * m, 2 * p)
            start_u(2 * m + 1, 2 * p + 1)

    for j in range(DEPTH):
        start_u(2 * j, 2 * j)
        start_u(2 * j + 1, 2 * j + 1)

    do_block(0, 0, True)
    start_u(2 * DEPTH, 0)
    start_u(2 * DEPTH + 1, 1)

    def outer(kk, carry):
        for p in range(DEPTH):
            m = DEPTH * kk + 1 + p
            do_block(m, (1 + p) % DEPTH, False)
            start_blk(m + DEPTH, (1 + p) % DEPTH)
        return carry

    n_mid = (blocks_per_w - 2) // DEPTH
    lax.fori_loop(0, n_mid, outer, 0)
    for j in range(1 + n_mid * DEPTH, blocks_per_w):
        do_block(j, j % DEPTH, False)
    wait_o(0)
    wait_o(1)


@functools.partial(jax.jit, static_argnums=(2, 3, 4))
def _sc_embed(idx3d, we, n_rows, l_len, d):
    blocks = n_rows // G
    blocks_per_w = blocks // NW
    blocks_per_l = n_rows // (l_len * G)
    assert blocks_per_w >= 2 * DEPTH + 2, blocks_per_w
    body = functools.partial(_sc_body, blocks_per_w=blocks_per_w,
                             blocks_per_l=blocks_per_l, l_len=l_len, d=d)
    k = pl.kernel(
        body,
        out_type=jax.ShapeDtypeStruct((n_rows, d), jnp.float32),
        mesh=plsc.VectorSubcoreMesh(core_axis_name="c", subcore_axis_name="s"),
        scratch_types=[
            pltpu.VMEM((2 * blocks_per_w, G), jnp.int32),
            pltpu.VMEM((2 * DEPTH, G, d), jnp.float32),
            pltpu.VMEM((2, H, d), jnp.float32),
            pltpu.VMEM((2, H), jnp.int32),
            pltpu.VMEM((H,), jnp.int32),
            [pltpu.SemaphoreType.DMA] * (2 * DEPTH),
            [pltpu.SemaphoreType.DMA] * 2,
        ],
    )
    return k(idx3d, we)


def kernel(inputs, we):
    b, l_len, two = inputs.shape
    n_rows = b * l_len
    d = we.shape[1]
    blocks_per_l = b // G
    x8 = inputs.astype(jnp.int32).reshape(blocks_per_l, G, l_len, two)
    wphys = jnp.transpose(x8, (2, 0, 3, 1))
    idx3d = wphys.reshape(NW, (n_rows * two) // (NW * G), G)
    out = _sc_embed(idx3d, we, n_rows, l_len, d)
    return out.reshape(b, l_len, d)


# speedup vs baseline: 13.8351x; 1.0156x over previous
import functools

import jax
import jax.numpy as jnp
from jax import lax
from jax.experimental import pallas as pl
from jax.experimental.pallas import tpu as pltpu
from jax.experimental.pallas import tpu_sc as plsc

NUM_CORES = 2
NUM_SUBCORES = 16
NW = NUM_CORES * NUM_SUBCORES
LANES = 16

G = 128


def _sc_body(idx_hbm, table_hbm, out_hbm, idx_v, rows_v, out_v, sidx_v,
             base_v, gsem0, gsem1, osem0, osem1,
             *, blocks_per_w, blocks_per_l, l_len, d, stage_rows):
    nsl = d // LANES
    wid = lax.axis_index("s") * NUM_CORES + lax.axis_index("c")
    blk0 = wid * blocks_per_w
    gsems = (gsem0, gsem1)
    osems = (osem0, osem1)

    row_lo = wid * (2 * blocks_per_w)
    astart = (row_lo // 8) * 8
    off = row_lo - astart
    pltpu.sync_copy(
        idx_hbm.at[pl.ds(pl.multiple_of(astart, 8), stage_rows)], idx_v)

    for s in range(G // LANES):
        base_v[pl.ds(s * LANES, LANES)] = (
            lax.iota(jnp.int32, LANES) + s * LANES) * l_len

    def start_g(j, b):
        pltpu.async_copy(table_hbm.at[idx_v.at[off + 2 * j]],
                         rows_v.at[b, pl.ds(0, G)], gsems[b])
        pltpu.async_copy(table_hbm.at[idx_v.at[off + 2 * j + 1]],
                         rows_v.at[b, pl.ds(G, G)], gsems[b])

    def wait_g(b):
        pltpu.make_async_copy(table_hbm.at[pl.ds(0, 2 * G)], rows_v.at[b],
                              gsems[b]).wait()

    def wait_o(b):
        pltpu.make_async_copy(out_v.at[b], out_hbm.at[sidx_v.at[b]],
                              osems[b]).wait()

    def compute_store(j, b, first):
        if not first:
            wait_o(b)
        blk = blk0 + j
        l = blk // blocks_per_l
        tc = blk - l * blocks_per_l
        cst = tc * (G * l_len) + l
        for s in range(G // LANES):
            sl = pl.ds(s * LANES, LANES)
            sidx_v[b, sl] = base_v[sl] + cst
        rref = rows_v.at[b]
        oref = out_v.at[b]

        @plsc.parallel_loop(0, G, unroll=4)
        def compute_row(c):
            for s in range(nsl):
                sl = pl.ds(s * LANES, LANES)
                oref[c, sl] = rref[c, sl] + rref[G + c, sl]

        pltpu.async_copy(oref, out_hbm.at[sidx_v.at[b]], osems[b])

    start_g(0, 0)
    start_g(1, 1)
    wait_g(0)
    compute_store(0, 0, True)
    start_g(2, 0)
    wait_g(1)
    compute_store(1, 1, True)

    def outer(kk, carry):
        j = 2 * kk
        start_g(j + 1, 1)
        wait_g(0)
        compute_store(j, 0, False)
        start_g(j + 2, 0)
        wait_g(1)
        compute_store(j + 1, 1, False)
        return carry

    lax.fori_loop(1, blocks_per_w // 2 - 1, outer, 0)
    start_g(blocks_per_w - 1, 1)
    wait_g(0)
    compute_store(blocks_per_w - 2, 0, False)
    wait_g(1)
    compute_store(blocks_per_w - 1, 1, False)
    wait_o(0)
    wait_o(1)


@functools.partial(jax.jit, static_argnums=(2, 3, 4))
def _sc_embed(idx2d, we, n_rows, l_len, d):
    blocks = (2 * n_rows) // (2 * G)
    blocks_per_w = blocks // NW
    blocks_per_l = n_rows // (l_len * G)
    assert blocks_per_w % 2 == 0 and blocks_per_w >= 6
    max_off = max((2 * blocks_per_w * w) % 8 for w in range(NW))
    stage_rows = 2 * blocks_per_w + max_off
    assert ((2 * blocks_per_w * (NW - 1)) // 8) * 8 + stage_rows \
        <= idx2d.shape[0]
    body = functools.partial(_sc_body, blocks_per_w=blocks_per_w,
                             blocks_per_l=blocks_per_l, l_len=l_len, d=d,
                             stage_rows=stage_rows)
    k = pl.kernel(
        body,
        out_type=jax.ShapeDtypeStruct((n_rows, d), jnp.float32),
        mesh=plsc.VectorSubcoreMesh(core_axis_name="c", subcore_axis_name="s"),
        scratch_types=[
            pltpu.VMEM((stage_rows, G), jnp.int32),
            pltpu.VMEM((2, 2 * G, d), jnp.float32),
            pltpu.VMEM((2, G, d), jnp.float32),
            pltpu.VMEM((2, G), jnp.int32),
            pltpu.VMEM((G,), jnp.int32),
            pltpu.SemaphoreType.DMA,
            pltpu.SemaphoreType.DMA,
            pltpu.SemaphoreType.DMA,
            pltpu.SemaphoreType.DMA,
        ],
    )
    return k(idx2d, we)


def kernel(inputs, we):
    b, l_len, two = inputs.shape
    n_rows = b * l_len
    d = we.shape[1]
    blocks_per_l = b // G
    x8 = inputs.astype(jnp.int32).reshape(blocks_per_l, G, l_len, two)
    wphys = jnp.transpose(x8, (2, 0, 3, 1))
    idx2d = wphys.reshape((n_rows * two) // G, G)
    out = _sc_embed(idx2d, we, n_rows, l_len, d)
    return out.reshape(b, l_len, d)
